# Initial kernel scaffold; baseline (speedup 1.0000x reference)
import jax
import jax.numpy as jnp
from jax.experimental import pallas as pl


def kernel(x, edge_index, batch_index, W1, b1, W2, b2, W3, b3, W4, b4, W5, b5, Wout, bout, Wout2, bout2):
    raise NotImplementedError("write your pallas kernel here")


# speedup vs baseline: 2.9729x; 2.9729x over previous
import functools

import jax
import jax.numpy as jnp
from jax import lax
from jax.experimental import pallas as pl
from jax.experimental.pallas import tpu as pltpu
from jax.experimental.pallas import tpu_sc as plsc

N_RAW = 10000
NP = 10240
NE_RAW = 160000
NG = 64
NW = 32
EB = 128
KB = 40
NE = NW * KB * EB
NSUB = 16
RPS = NP // NSUB
PB = 128
MB = 1280
C = 128


def _sc_scatter_pass(nchunks):
    mesh = plsc.VectorSubcoreMesh(core_axis_name="c", subcore_axis_name="s")

    @functools.partial(
        pl.kernel,
        mesh=mesh,
        out_type=jax.ShapeDtypeStruct((nchunks * 2 * NP, C), jnp.float32),
        scratch_types=[
            pltpu.VMEM((KB, EB), jnp.int32),
            pltpu.VMEM((KB, EB), jnp.int32),
            pltpu.VMEM((EB, C), jnp.float32),
            pltpu.VMEM((EB, C), jnp.float32),
            pltpu.VMEM_SHARED((NP, C), jnp.float32),
            pltpu.SemaphoreType.DMA,
        ],
    )
    def sc_kernel(table_h, srcw_h, dstw_h, zeros_h, out_h,
                  sidx, didx, gb0, gb1, acc, gsem):
        core = lax.axis_index("c")
        sub = lax.axis_index("s")
        wid = sub * 2 + core
        for ch in range(nchunks):
            pltpu.sync_copy(srcw_h.at[ch * NW + wid], sidx)
            pltpu.sync_copy(dstw_h.at[ch * NW + wid], didx)
            pltpu.sync_copy(zeros_h, gb0)
            for p in range(RPS // PB):
                pltpu.sync_copy(gb0, acc.at[pl.ds(sub * RPS + p * PB, PB)])
            plsc.subcore_barrier()

            pltpu.async_copy(table_h.at[sidx.at[0]], gb0, gsem)

            def step(g, carry):
                for b in range(2):
                    j = g * 2 + b
                    buf = gb0 if b == 0 else gb1
                    nbuf = gb1 if b == 0 else gb0
                    pltpu.make_async_copy(
                        table_h.at[sidx.at[j]], buf, gsem).wait()

                    @pl.when(j + 1 < KB)
                    def _():
                        pltpu.async_copy(table_h.at[sidx.at[j + 1]], nbuf, gsem)

                    pltpu.sync_copy(buf, acc.at[didx.at[j]], add=True)
                return carry

            lax.fori_loop(0, KB // 2, step, 0)
            plsc.subcore_barrier()
            for p in range(RPS // PB):
                off = sub * RPS + p * PB
                pltpu.sync_copy(acc.at[pl.ds(off, PB)], gb0)
                pltpu.sync_copy(
                    gb0, out_h.at[pl.ds((ch * 2 + core) * NP + off, PB)])

    return sc_kernel


def _tc_first_body(deg_ref, x_ref, w_ref, yp_ref, dinv_ref):
    deg = deg_ref[0] + deg_ref[1] + 1.0
    rows = lax.broadcasted_iota(jnp.int32, (NP, 1), 0)
    dinv = jnp.where(rows < N_RAW, lax.rsqrt(deg), 0.0)
    y = jnp.dot(x_ref[...], w_ref[...], preferred_element_type=jnp.float32)
    yp_ref[0] = y[:, :C] * dinv
    yp_ref[1] = y[:, C:] * dinv
    dinv_ref[...] = dinv


def _tc_first(degcol, xp, w1p):
    return pl.pallas_call(
        _tc_first_body,
        out_shape=(jax.ShapeDtypeStruct((2, NP, C), jnp.float32),
                   jax.ShapeDtypeStruct((NP, 1), jnp.float32)),
    )(degcol, xp, w1p)


def _tc_layer_body(n_in, parts_ref, yp_ref, dinv_ref, b_ref, w_ref, out_ref):
    j = pl.program_id(2)
    act = jnp.maximum(
        dinv_ref[...] * (parts_ref[0, 0] + parts_ref[0, 1] + yp_ref[0])
        + b_ref[0], 0.0)
    part = jnp.dot(act, w_ref[0, 0], preferred_element_type=jnp.float32)

    @pl.when(j == 0)
    def _():
        out_ref[0] = part

    @pl.when(j > 0)
    def _():
        out_ref[0] += part

    @pl.when(j == n_in - 1)
    def _():
        out_ref[0] *= dinv_ref[...]


def _tc_layer(parts, yp, dinv, bpad, wpad, n_in, n_out):
    grid = (NP // MB, n_out, n_in)
    return pl.pallas_call(
        functools.partial(_tc_layer_body, n_in),
        grid=grid,
        in_specs=[
            pl.BlockSpec((1, 2, MB, C), lambda m, k, j: (j, 0, m, 0)),
            pl.BlockSpec((1, MB, C), lambda m, k, j: (j, m, 0)),
            pl.BlockSpec((MB, 1), lambda m, k, j: (m, 0)),
            pl.BlockSpec((1, 1, C), lambda m, k, j: (j, 0, 0)),
            pl.BlockSpec((1, 1, C, C), lambda m, k, j: (j, k, 0, 0)),
        ],
        out_specs=pl.BlockSpec((1, MB, C), lambda m, k, j: (k, m, 0)),
        out_shape=jax.ShapeDtypeStruct((n_out, NP, C), jnp.float32),
    )(parts, yp, dinv, bpad, wpad)


def _tc_pool_body(parts_ref, yp_ref, dinv_ref, b_ref, batch_ref,
                  pooled_ref, cnt_ref):
    act = jnp.maximum(
        dinv_ref[...] * (parts_ref[0, 0] + parts_ref[0, 1] + yp_ref[0])
        + b_ref[0], 0.0)
    seg = lax.broadcasted_iota(jnp.int32, (NG, NP), 0)
    p = (batch_ref[...] == seg).astype(jnp.float32)
    pooled_ref[0] = jnp.dot(p, act, preferred_element_type=jnp.float32)
    cnt_ref[...] = jnp.sum(p, axis=1, keepdims=True)


def _tc_pool(parts, yp, dinv, bpad, batch2d):
    return pl.pallas_call(
        _tc_pool_body,
        grid=(2,),
        in_specs=[
            pl.BlockSpec((1, 2, NP, C), lambda j: (j, 0, 0, 0)),
            pl.BlockSpec((1, NP, C), lambda j: (j, 0, 0)),
            pl.BlockSpec((NP, 1), lambda j: (0, 0)),
            pl.BlockSpec((1, 1, C), lambda j: (j, 0, 0)),
            pl.BlockSpec((1, NP), lambda j: (0, 0)),
        ],
        out_specs=(pl.BlockSpec((1, NG, C), lambda j: (j, 0, 0)),
                   pl.BlockSpec((NG, 1), lambda j: (0, 0))),
        out_shape=(jax.ShapeDtypeStruct((2, NG, C), jnp.float32),
                   jax.ShapeDtypeStruct((NG, 1), jnp.float32)),
    )(parts, yp, dinv, bpad, batch2d)


def _tc_mlp_body(pooled_ref, cnt_ref, wo_ref, bo_ref, wo2_ref, bo2_ref,
                 out_ref):
    inv = 1.0 / jnp.maximum(cnt_ref[...], 1.0)
    h = bo_ref[...]
    for j in range(2):
        h = h + jnp.dot(pooled_ref[j] * inv, wo_ref[j],
                        preferred_element_type=jnp.float32)
    h = jnp.maximum(h, 0.0)
    o = jnp.dot(h, wo2_ref[...], preferred_element_type=jnp.float32)
    out_ref[...] = o[:, 0:1] + bo2_ref[...]


def _tc_mlp(pooled, cnt, wop, bop, wo2p, bo2p):
    return pl.pallas_call(
        _tc_mlp_body,
        out_shape=jax.ShapeDtypeStruct((NG, 1), jnp.float32),
    )(pooled, cnt, wop, bop, wo2p, bo2p)


def _pad2(a, r, c):
    return jnp.pad(a, ((0, r - a.shape[0]), (0, c - a.shape[1])))


def kernel(x, edge_index, batch_index, W1, b1, W2, b2, W3, b3, W4, b4,
           W5, b5, Wout, bout, Wout2, bout2):
    f32 = jnp.float32
    src = edge_index[0].astype(jnp.int32)
    dst = edge_index[1].astype(jnp.int32)
    padn = NE - NE_RAW
    srcp = jnp.concatenate([src, jnp.full((padn,), N_RAW, jnp.int32)])
    dstp = jnp.concatenate([dst, jnp.full((padn,), N_RAW, jnp.int32)])
    src_b = srcp.reshape(NW, KB, EB)
    dst_b = dstp.reshape(NW, KB, EB)
    src_c = {n: jnp.concatenate([src_b + k * NP for k in range(n)])
             for n in (1, 2, 3, 4)}
    dst_c = {n: jnp.concatenate([dst_b] * n) for n in (1, 2, 3, 4)}

    zc = jnp.zeros((PB, C), f32)
    ones_t = jnp.concatenate(
        [jnp.ones((N_RAW, C), f32), jnp.zeros((NP - N_RAW, C), f32)])

    n_ins = [2, 3, 4, 3, 2]
    xp = _pad2(x, NP, 48)
    w1p = _pad2(W1, 48, 2 * C)
    wnext = []
    for w, n_in, n_out in ((W2, 2, 3), (W3, 3, 4), (W4, 4, 3), (W5, 3, 2)):
        wp = _pad2(w, n_in * C, n_out * C)
        wnext.append(wp.reshape(n_in, C, n_out, C).transpose(0, 2, 1, 3))
    bp = [jnp.pad(b, (0, n * C - b.shape[0])).reshape(n, 1, C)
          for b, n in ((b1, 2), (b2, 3), (b3, 4), (b4, 3), (b5, 2))]
    wop = _pad2(Wout, 2 * C, C).reshape(2, C, C)
    wo2p = _pad2(Wout2, C, C)
    bop = jnp.pad(bout, (0, C - 75)).reshape(1, C)
    bo2p = jnp.pad(bout2, (0, C - 1)).reshape(1, C)[:, 0:1]
    batch2d = jnp.concatenate(
        [batch_index.astype(jnp.int32), jnp.full((NP - N_RAW,), NG, jnp.int32)]
    ).reshape(1, NP)

    deg_flat = _sc_scatter_pass(1)(ones_t, src_c[1], dst_c[1], zc)
    degcol = deg_flat.reshape(2, NP, C)[:, :, 0:1]

    yp, dinv = _tc_first(degcol, xp, w1p)

    for li in range(5):
        n_in = n_ins[li]
        table = yp.reshape(n_in * NP, C)
        parts = _sc_scatter_pass(n_in)(table, src_c[n_in], dst_c[n_in], zc)
        parts = parts.reshape(n_in, 2, NP, C)
        if li < 4:
            yp = _tc_layer(parts, yp, dinv, bp[li], wnext[li],
                           n_in, n_ins[li + 1])
        else:
            pooled, cnt = _tc_pool(parts, yp, dinv, bp[4], batch2d)
    return _tc_mlp(pooled, cnt, wop, bop, wo2p, bo2p)
